# baseline (speedup 1.0000x reference)
import jax
import jax.numpy as jnp
from jax.experimental import pallas as pl

EPS = 1e-5
_BLK = 4000

_HI = jax.lax.Precision.HIGHEST


def _stats_kernel(xyz_ref, feat_ref, s3_ref, s13_ref, c33_ref, c3f_ref, cff_ref):
    i = pl.program_id(0)
    a = xyz_ref[...]
    b = feat_ref[...]
    s3 = jnp.sum(a, axis=0, keepdims=True)
    s13 = jnp.sum(b, axis=0, keepdims=True)
    dn = (((0,), (0,)), ((), ()))
    c33 = jax.lax.dot_general(a, a, dn, precision=_HI, preferred_element_type=jnp.float32)
    c3f = jax.lax.dot_general(a, b, dn, precision=_HI, preferred_element_type=jnp.float32)
    cff = jax.lax.dot_general(b, b, dn, precision=_HI, preferred_element_type=jnp.float32)

    @pl.when(i == 0)
    def _init():
        s3_ref[...] = s3
        s13_ref[...] = s13
        c33_ref[...] = c33
        c3f_ref[...] = c3f
        cff_ref[...] = cff

    @pl.when(i != 0)
    def _acc():
        s3_ref[...] += s3
        s13_ref[...] += s13
        c33_ref[...] += c33
        c3f_ref[...] += c3f
        cff_ref[...] += cff


def _apply_kernel(xyz_ref, feat_ref, s3w_ref, s13w_ref, shift_ref, out_ref):
    a = xyz_ref[...]
    b = feat_ref[...]
    p = jax.lax.dot_general(a, s3w_ref[...], (((1,), (0,)), ((), ())),
                            precision=_HI, preferred_element_type=jnp.float32)
    p = p + jax.lax.dot_general(b, s13w_ref[...], (((1,), (0,)), ((), ())),
                                precision=_HI, preferred_element_type=jnp.float32)
    out_ref[...] = jnp.maximum(p + shift_ref[...], 0.0)


def kernel(xyz, feat, offset, velocities, W, gamma, beta):
    n = xyz.shape[0]
    f = feat.shape[1]
    blk = _BLK if n % _BLK == 0 else n
    nb = n // blk

    row = lambda i: (i, 0)
    zero = lambda i: (0, 0)
    s3, s13, c33, c3f, cff = pl.pallas_call(
        _stats_kernel,
        grid=(nb,),
        in_specs=[
            pl.BlockSpec((blk, 3), row),
            pl.BlockSpec((blk, f), row),
        ],
        out_specs=[
            pl.BlockSpec((1, 3), zero),
            pl.BlockSpec((1, f), zero),
            pl.BlockSpec((3, 3), zero),
            pl.BlockSpec((3, f), zero),
            pl.BlockSpec((f, f), zero),
        ],
        out_shape=[
            jax.ShapeDtypeStruct((1, 3), jnp.float32),
            jax.ShapeDtypeStruct((1, f), jnp.float32),
            jax.ShapeDtypeStruct((3, 3), jnp.float32),
            jax.ShapeDtypeStruct((3, f), jnp.float32),
            jax.ShapeDtypeStruct((f, f), jnp.float32),
        ],
    )(xyz, feat)

    s = jnp.concatenate([s3[0], s13[0]])
    c = jnp.block([[c33, c3f], [c3f.T, cff]])
    mean = W @ (s / n)
    m = W @ (c / n)
    e2 = jnp.sum(m * W, axis=1)
    var = e2 - mean * mean
    scale = gamma / jnp.sqrt(var + EPS)
    shift = (beta - mean * scale)[None, :]
    sw = W.T * scale[None, :]
    s3w, s13w = sw[:3], sw[3:]

    new_feat = pl.pallas_call(
        _apply_kernel,
        grid=(nb,),
        in_specs=[
            pl.BlockSpec((blk, 3), row),
            pl.BlockSpec((blk, f), row),
            pl.BlockSpec((3, 16), zero),
            pl.BlockSpec((f, 16), zero),
            pl.BlockSpec((1, 16), zero),
        ],
        out_specs=pl.BlockSpec((blk, 16), row),
        out_shape=jax.ShapeDtypeStruct((n, 16), jnp.float32),
    )(xyz, feat, s3w, s13w, shift)

    return (xyz, new_feat, offset, feat)


# speedup vs baseline: 8.1190x; 8.1190x over previous
import jax
import jax.numpy as jnp
from jax.experimental import pallas as pl
from jax.experimental.pallas import tpu as pltpu

EPS = 1e-5
_B = 4096


def _fused_kernel(n_ref, xyzT_ref, featT_ref, w3_ref, wf_ref, g_ref, b_ref,
                  outT_ref, velT_ref, stash_ref, s_ref, q_ref, sc_ref, sh_ref):
    p_id = pl.program_id(0)
    i = pl.program_id(1)
    nb = pl.num_programs(1)
    n = n_ref[0]

    @pl.when(p_id == 0)
    def _phase0():
        a = xyzT_ref[...]
        f = featT_ref[...]
        velT_ref[...] = f
        dn = (((1,), (0,)), ((), ()))
        p = jax.lax.dot_general(w3_ref[...], a, dn,
                                preferred_element_type=jnp.float32)
        p = p + jax.lax.dot_general(wf_ref[...], f, dn,
                                    preferred_element_type=jnp.float32)
        valid = n - i * _B
        lane = jax.lax.broadcasted_iota(jnp.int32, (16, _B), 1)
        pm = jnp.where(lane < valid, p, 0.0)
        ps = jnp.sum(pm, axis=1, keepdims=True)
        pq = jnp.sum(pm * pm, axis=1, keepdims=True)
        stash_ref[:, pl.ds(i * _B, _B)] = p.astype(jnp.bfloat16)

        @pl.when(i == 0)
        def _():
            s_ref[...] = ps
            q_ref[...] = pq

        @pl.when(i != 0)
        def _():
            s_ref[...] += ps
            q_ref[...] += pq

        @pl.when(i == nb - 1)
        def _():
            nf = n.astype(jnp.float32)
            mean = s_ref[...] / nf
            var = q_ref[...] / nf - mean * mean
            scale = g_ref[...] * jax.lax.rsqrt(var + EPS)
            sc_ref[...] = scale
            sh_ref[...] = b_ref[...] - mean * scale

    @pl.when(p_id == 1)
    def _phase1():
        p = stash_ref[:, pl.ds(i * _B, _B)].astype(jnp.float32)
        outT_ref[...] = jnp.maximum(p * sc_ref[...] + sh_ref[...], 0.0)


def kernel(xyz, feat, offset, velocities, W, gamma, beta):
    n = xyz.shape[0]
    nb = pl.cdiv(n, _B)
    xyzT = xyz.T
    featT = feat.T
    w3 = W[:, :3]
    wf = W[:, 3:]
    g = gamma.reshape(16, 1)
    b = beta.reshape(16, 1)
    n_arr = jnp.full((1,), n, dtype=jnp.int32)

    const = lambda p, i: (0, 0)
    stream = lambda p, i: (0, jnp.where(p == 0, i, 0))
    hold_last = lambda p, i: (0, jnp.where(p == 0, i, nb - 1))
    outmap = lambda p, i: (0, jnp.where(p == 0, 0, i))

    outT, velT = pl.pallas_call(
        _fused_kernel,
        grid=(2, nb),
        in_specs=[
            pl.BlockSpec(memory_space=pltpu.SMEM),
            pl.BlockSpec((3, _B), stream),
            pl.BlockSpec((13, _B), stream),
            pl.BlockSpec((16, 3), const),
            pl.BlockSpec((16, 13), const),
            pl.BlockSpec((16, 1), const),
            pl.BlockSpec((16, 1), const),
        ],
        out_specs=[
            pl.BlockSpec((16, _B), outmap),
            pl.BlockSpec((13, _B), hold_last),
        ],
        out_shape=[
            jax.ShapeDtypeStruct((16, n), jnp.float32),
            jax.ShapeDtypeStruct((13, n), jnp.float32),
        ],
        compiler_params=pltpu.CompilerParams(
            vmem_limit_bytes=64 * 1024 * 1024,
        ),
        scratch_shapes=[
            pltpu.VMEM((16, nb * _B), jnp.bfloat16),
            pltpu.VMEM((16, 1), jnp.float32),
            pltpu.VMEM((16, 1), jnp.float32),
            pltpu.VMEM((16, 1), jnp.float32),
            pltpu.VMEM((16, 1), jnp.float32),
        ],
    )(n_arr, xyzT, featT, w3, wf, g, b)

    return (xyz, outT.T, offset, velT.T)


# speedup vs baseline: 16.8530x; 2.0758x over previous
import jax
import jax.numpy as jnp
from jax.experimental import pallas as pl
from jax.experimental.pallas import tpu as pltpu

EPS = 1e-5
_B = 16384


def _fused_kernel(n_ref, xyzT_ref, featT_ref, w3_ref, wf_ref, g_ref, b_ref,
                  outT_ref, velT_ref, stash_ref, s_ref, q_ref, sc_ref, sh_ref):
    p_id = pl.program_id(0)
    i = pl.program_id(1)
    nb = pl.num_programs(1)
    n = n_ref[0]

    @pl.when(p_id == 0)
    def _phase0():
        a = xyzT_ref[...]
        f = featT_ref[...]
        velT_ref[...] = f
        dn = (((1,), (0,)), ((), ()))
        p = jax.lax.dot_general(w3_ref[...], a, dn,
                                preferred_element_type=jnp.float32)
        p = p + jax.lax.dot_general(wf_ref[...], f, dn,
                                    preferred_element_type=jnp.float32)
        stash_ref[:, pl.ds(i * _B, _B)] = p.astype(jnp.bfloat16)

        full = jnp.logical_or(i != nb - 1, n == nb * _B)
        pm = jax.lax.cond(
            full,
            lambda p_: p_,
            lambda p_: jnp.where(
                jax.lax.broadcasted_iota(jnp.int32, (16, _B), 1) < n - i * _B,
                p_, 0.0),
            p,
        )
        ps = jnp.sum(pm, axis=1, keepdims=True)
        pq = jnp.sum(pm * pm, axis=1, keepdims=True)

        @pl.when(i == 0)
        def _():
            s_ref[...] = ps
            q_ref[...] = pq

        @pl.when(i != 0)
        def _():
            s_ref[...] += ps
            q_ref[...] += pq

        @pl.when(i == nb - 1)
        def _():
            nf = n.astype(jnp.float32)
            mean = s_ref[...] / nf
            var = q_ref[...] / nf - mean * mean
            scale = g_ref[...] * jax.lax.rsqrt(var + EPS)
            sc_ref[...] = scale
            sh_ref[...] = b_ref[...] - mean * scale

    @pl.when(p_id == 1)
    def _phase1():
        p = stash_ref[:, pl.ds(i * _B, _B)].astype(jnp.float32)
        outT_ref[...] = jnp.maximum(p * sc_ref[...] + sh_ref[...], 0.0)


def kernel(xyz, feat, offset, velocities, W, gamma, beta):
    n = xyz.shape[0]
    nb = pl.cdiv(n, _B)
    xyzT = xyz.T
    featT = feat.T
    w3 = W[:, :3]
    wf = W[:, 3:]
    g = gamma.reshape(16, 1)
    b = beta.reshape(16, 1)
    n_arr = jnp.full((1,), n, dtype=jnp.int32)

    const = lambda p, i: (0, 0)
    stream = lambda p, i: (0, jnp.where(p == 0, i, 0))
    hold_last = lambda p, i: (0, jnp.where(p == 0, i, nb - 1))
    outmap = lambda p, i: (0, jnp.where(p == 0, 0, i))

    outT, velT = pl.pallas_call(
        _fused_kernel,
        grid=(2, nb),
        in_specs=[
            pl.BlockSpec(memory_space=pltpu.SMEM),
            pl.BlockSpec((3, _B), stream),
            pl.BlockSpec((13, _B), stream),
            pl.BlockSpec((16, 3), const),
            pl.BlockSpec((16, 13), const),
            pl.BlockSpec((16, 1), const),
            pl.BlockSpec((16, 1), const),
        ],
        out_specs=[
            pl.BlockSpec((16, _B), outmap),
            pl.BlockSpec((13, _B), hold_last),
        ],
        out_shape=[
            jax.ShapeDtypeStruct((16, n), jnp.float32),
            jax.ShapeDtypeStruct((13, n), jnp.float32),
        ],
        compiler_params=pltpu.CompilerParams(
            vmem_limit_bytes=100 * 1024 * 1024,
        ),
        scratch_shapes=[
            pltpu.VMEM((16, nb * _B), jnp.bfloat16),
            pltpu.VMEM((16, 1), jnp.float32),
            pltpu.VMEM((16, 1), jnp.float32),
            pltpu.VMEM((16, 1), jnp.float32),
            pltpu.VMEM((16, 1), jnp.float32),
        ],
    )(n_arr, xyzT, featT, w3, wf, g, b)

    return (xyz, outT.T, offset, velT.T)
